# baseline (speedup 1.0000x reference)
import functools

import jax
import jax.numpy as jnp
from jax import lax
from jax.experimental import pallas as pl
from jax.experimental.pallas import tpu as pltpu
from jax.experimental.pallas import tpu_sc as plsc

_NUM_CLASSES = 1000
_C_PAD = 1024
_N = 320000
_D = 128
_SPLIT = 160000
_CHUNK = 256
_SUB = 128
_NSUB = _CHUNK // _SUB
_NCHUNKS = _N // _CHUNK
_SPLIT_CH = _SPLIT // _CHUNK
_NSC_CH = _NCHUNKS - _SPLIT_CH
_NC = 2
_NS = 16
_NW = _NC * _NS
_CPW = _NSC_CH // _NW
_REM = _NSC_CH % _NW
_MAX_CPW = _CPW + (1 if _REM else 0)
_NROUNDS = -(-_MAX_CPW // 3)
_ROWS_PER_TILE = _C_PAD // _NS

_TBLK = 128
_NTBLK = _SPLIT // _TBLK
_W = 136
_OUTROWS = 1136

_mesh = plsc.VectorSubcoreMesh(core_axis_name="c", subcore_axis_name="s")


@functools.partial(
    pl.kernel,
    out_type=(
        jax.ShapeDtypeStruct((_NC, _C_PAD, _D), jnp.float32),
        jax.ShapeDtypeStruct((_NW, _C_PAD), jnp.float32),
    ),
    mesh=_mesh,
    compiler_params=pltpu.CompilerParams(needs_layout_passes=False),
    scratch_types=[
        pltpu.VMEM((_CHUNK, _D), jnp.float32),
        pltpu.VMEM((_CHUNK, _D), jnp.float32),
        pltpu.VMEM((_CHUNK, _D), jnp.float32),
        pltpu.VMEM((_SUB,), jnp.int32),
        pltpu.VMEM((_SUB,), jnp.int32),
        pltpu.VMEM((_SUB,), jnp.int32),
        pltpu.VMEM((_SUB,), jnp.int32),
        pltpu.VMEM((_SUB,), jnp.int32),
        pltpu.VMEM((_SUB,), jnp.int32),
        pltpu.VMEM((_C_PAD,), jnp.float32),
        pltpu.VMEM_SHARED((_C_PAD, _D), jnp.float32),
        pltpu.SemaphoreType.DMA,
        pltpu.SemaphoreType.DMA,
        pltpu.SemaphoreType.DMA,
        pltpu.SemaphoreType.DMA,
        pltpu.SemaphoreType.DMA,
        pltpu.SemaphoreType.DMA,
    ],
)
def _segment_sums(feat, lab1d, zsum, psums, pcnts,
                  rows0, rows1, rows2,
                  idx00, idx01, idx10, idx11, idx20, idx21,
                  cnt_v, acc_s,
                  lsem0, lsem1, lsem2, ssem0, ssem1, ssem2):
    rows = (rows0, rows1, rows2)
    idx = ((idx00, idx01), (idx10, idx11), (idx20, idx21))
    lsem = (lsem0, lsem1, lsem2)
    ssem = (ssem0, ssem1, ssem2)

    c = lax.axis_index("c")
    s = lax.axis_index("s")
    w = s * _NC + c
    jbase = _SPLIT_CH + _CPW * w + jnp.minimum(w, _REM)
    jend = _SPLIT_CH + _CPW * (w + 1) + jnp.minimum(w + 1, _REM)
    nch = jend - jbase

    base = s * _ROWS_PER_TILE
    pltpu.sync_copy(zsum.at[pl.ds(base, _ROWS_PER_TILE)],
                    acc_s.at[pl.ds(base, _ROWS_PER_TILE)])

    zv = jnp.zeros((16,), jnp.float32)

    def zbody(k, carry):
        cnt_v[pl.ds(k * 16, 16)] = zv
        return carry

    lax.fori_loop(0, _C_PAD // 16, zbody, 0)
    plsc.subcore_barrier()

    ones16 = jnp.ones((16,), jnp.float32)

    def start_load(r, j):
        pltpu.async_copy(feat.at[pl.ds(j * _CHUNK, _CHUNK)], rows[r], lsem[r])
        for k in range(_NSUB):
            pltpu.async_copy(lab1d.at[pl.ds(j * _CHUNK + k * _SUB, _SUB)],
                             idx[r][k], lsem[r])

    def wait_load(r, j):
        pltpu.make_async_copy(feat.at[pl.ds(j * _CHUNK, _CHUNK)],
                              rows[r], lsem[r]).wait()
        for k in range(_NSUB):
            pltpu.make_async_copy(
                lab1d.at[pl.ds(j * _CHUNK + k * _SUB, _SUB)],
                idx[r][k], lsem[r]).wait()

    def start_scatter(r):
        for k in range(_NSUB):
            pltpu.async_copy(rows[r].at[pl.ds(k * _SUB, _SUB)],
                             acc_s.at[idx[r][k]], ssem[r], add=True)

    def wait_scatter(r):
        for k in range(_NSUB):
            pltpu.make_async_copy(rows[r].at[pl.ds(k * _SUB, _SUB)],
                                  acc_s.at[idx[r][k]], ssem[r]).wait()

    def do_counts(r):
        for k in range(_NSUB):
            for m in range(_SUB // 16):
                iv = idx[r][k][pl.ds(m * 16, 16)]
                plsc.addupdate_scatter(cnt_v, [iv], ones16)

    def step(j, r, has_prev):
        @pl.when(j < jend)
        def _():
            wait_load(r, j)
            start_scatter(r)
            do_counts(r)
            rp = (r - 1) % 3
            if has_prev:
                wait_scatter(rp)
            jn = j + 2

            @pl.when(jn < jend)
            def _():
                start_load(rp, jn)

    for r in range(2):
        jj = jbase + r

        @pl.when(jj < jend)
        def _():
            start_load(r, jj)

    for r in range(3):
        step(jbase + r, r, has_prev=(r != 0))

    def body(i, carry):
        for r in range(3):
            step(jbase + 3 * i + r, r, has_prev=True)
        return carry

    lax.fori_loop(1, _NROUNDS, body, 0)

    for r in range(3):
        @pl.when(lax.rem(nch - 1, 3) == r)
        def _():
            wait_scatter(r)

    plsc.subcore_barrier()
    pltpu.sync_copy(acc_s.at[pl.ds(base, _ROWS_PER_TILE)],
                    psums.at[c, pl.ds(base, _ROWS_PER_TILE)])
    pltpu.sync_copy(cnt_v, pcnts.at[w])


def _tc_body(starts_ref, lab_ref, feat_ref, out_ref):
    i = pl.program_id(0)

    @pl.when(i == 0)
    def _():
        out_ref[...] = jnp.zeros_like(out_ref)

    astart = pl.multiple_of(starts_ref[i], 8)
    labs = lab_ref[0]
    feats = feat_ref[...]
    xp = jnp.concatenate(
        [feats, jnp.ones((_TBLK, _W - _D), jnp.float32)], axis=1
    ).astype(jnp.bfloat16)
    local = labs - astart
    span_ok = jnp.max(local) < _W

    @pl.when(span_ok)
    def _fast():
        iota = lax.broadcasted_iota(jnp.int32, (_W, _TBLK), 0)
        oh = (iota == jnp.broadcast_to(local, (_W, _TBLK))).astype(
            jnp.bfloat16)
        contrib = lax.dot_general(oh, xp, (((1,), (0,)), ((), ())),
                                  preferred_element_type=jnp.float32)
        cur = out_ref[pl.ds(astart, _W), :]
        out_ref[pl.ds(astart, _W), :] = cur + contrib

    @pl.when(jnp.logical_not(span_ok))
    def _slow():
        iota = lax.broadcasted_iota(jnp.int32, (_OUTROWS, _TBLK), 0)
        oh = (iota == jnp.broadcast_to(labs, (_OUTROWS, _TBLK))).astype(
            jnp.bfloat16)
        contrib = lax.dot_general(oh, xp, (((1,), (0,)), ((), ())),
                                  preferred_element_type=jnp.float32)
        out_ref[...] = out_ref[...] + contrib


_tc_partial = pl.pallas_call(
    _tc_body,
    grid_spec=pltpu.PrefetchScalarGridSpec(
        num_scalar_prefetch=1,
        grid=(_NTBLK,),
        in_specs=[
            pl.BlockSpec((1, 1, _TBLK), lambda i, s: (i, 0, 0)),
            pl.BlockSpec((_TBLK, _D), lambda i, s: (i, 0)),
        ],
        out_specs=pl.BlockSpec((_OUTROWS, _W), lambda i, s: (0, 0)),
    ),
    out_shape=jax.ShapeDtypeStruct((_OUTROWS, _W), jnp.float32),
)


def _combine_body(ps_ref, pc_ref, tc_ref, o_ref):
    sums = ps_ref[0] + ps_ref[1] + tc_ref[: _C_PAD, : _D]
    cnts = jnp.sum(pc_ref[...], axis=0) + tc_ref[: _C_PAD, _D]
    denom = jnp.maximum(cnts[:, None], 1.0)
    o_ref[...] = (sums / denom)[:_NUM_CLASSES]


def kernel(context_features, context_labels):
    labels = context_labels.astype(jnp.int32)
    zsum = jnp.zeros((_C_PAD, _D), jnp.float32)
    psums, pcnts = _segment_sums(context_features, labels, zsum)

    lab3d = labels[:_SPLIT].reshape(_NTBLK, 1, _TBLK)
    starts = (labels[:_SPLIT:_TBLK] // 8) * 8
    tc_part = _tc_partial(starts, lab3d, context_features[:_SPLIT])

    return pl.pallas_call(
        _combine_body,
        out_shape=jax.ShapeDtypeStruct((_NUM_CLASSES, _D), jnp.float32),
    )(psums, pcnts, tc_part)


# speedup vs baseline: 7.2331x; 7.2331x over previous
import functools

import jax
import jax.numpy as jnp
from jax import lax
from jax.experimental import pallas as pl
from jax.experimental.pallas import tpu as pltpu
from jax.experimental.pallas import tpu_sc as plsc

_NUM_CLASSES = 1000
_C_PAD = 1024
_N = 320000
_D = 128
_CHUNK = 256
_SUB = 128
_NSUB = _CHUNK // _SUB
_NCHUNKS = _N // _CHUNK
_NC = 2
_NS = 16
_NW = _NC * _NS
_CPW = _NCHUNKS // _NW
_MAX_CPW = _CPW + 1
_NROUNDS = -(-_MAX_CPW // 3)
_ROWS_PER_TILE = _C_PAD // _NS

_mesh = plsc.VectorSubcoreMesh(core_axis_name="c", subcore_axis_name="s")


@functools.partial(
    pl.kernel,
    out_type=(
        jax.ShapeDtypeStruct((_NC, _C_PAD, _D), jnp.float32),
        jax.ShapeDtypeStruct((_NW, _C_PAD), jnp.float32),
    ),
    mesh=_mesh,
    compiler_params=pltpu.CompilerParams(needs_layout_passes=False),
    scratch_types=[
        pltpu.VMEM((_CHUNK, _D), jnp.float32),
        pltpu.VMEM((_CHUNK, _D), jnp.float32),
        pltpu.VMEM((_CHUNK, _D), jnp.float32),
        pltpu.VMEM((_SUB,), jnp.int32),
        pltpu.VMEM((_SUB,), jnp.int32),
        pltpu.VMEM((_SUB,), jnp.int32),
        pltpu.VMEM((_SUB,), jnp.int32),
        pltpu.VMEM((_SUB,), jnp.int32),
        pltpu.VMEM((_SUB,), jnp.int32),
        pltpu.VMEM((_C_PAD,), jnp.float32),
        pltpu.VMEM_SHARED((_C_PAD, _D), jnp.float32),
        pltpu.SemaphoreType.DMA,
        pltpu.SemaphoreType.DMA,
        pltpu.SemaphoreType.DMA,
        pltpu.SemaphoreType.DMA,
        pltpu.SemaphoreType.DMA,
        pltpu.SemaphoreType.DMA,
    ],
)
def _segment_sums(feat, lab1d, zsum, psums, pcnts,
                  rows0, rows1, rows2,
                  idx00, idx01, idx10, idx11, idx20, idx21,
                  cnt_v, acc_s,
                  lsem0, lsem1, lsem2, ssem0, ssem1, ssem2):
    rows = (rows0, rows1, rows2)
    idx = ((idx00, idx01), (idx10, idx11), (idx20, idx21))
    lsem = (lsem0, lsem1, lsem2)
    ssem = (ssem0, ssem1, ssem2)

    c = lax.axis_index("c")
    s = lax.axis_index("s")
    w = s * _NC + c
    jbase = _CPW * w + jnp.minimum(w, 2)
    jend = _CPW * (w + 1) + jnp.minimum(w + 1, 2)
    nch = jend - jbase

    base = s * _ROWS_PER_TILE
    pltpu.sync_copy(zsum.at[pl.ds(base, _ROWS_PER_TILE)],
                    acc_s.at[pl.ds(base, _ROWS_PER_TILE)])

    zv = jnp.zeros((16,), jnp.float32)

    def zbody(k, carry):
        cnt_v[pl.ds(k * 16, 16)] = zv
        return carry

    lax.fori_loop(0, _C_PAD // 16, zbody, 0)
    plsc.subcore_barrier()

    ones16 = jnp.ones((16,), jnp.float32)

    def start_load(r, j):
        pltpu.async_copy(feat.at[pl.ds(j * _CHUNK, _CHUNK)], rows[r], lsem[r])
        for k in range(_NSUB):
            pltpu.async_copy(lab1d.at[pl.ds(j * _CHUNK + k * _SUB, _SUB)],
                             idx[r][k], lsem[r])

    def wait_load(r, j):
        pltpu.make_async_copy(feat.at[pl.ds(j * _CHUNK, _CHUNK)],
                              rows[r], lsem[r]).wait()
        for k in range(_NSUB):
            pltpu.make_async_copy(
                lab1d.at[pl.ds(j * _CHUNK + k * _SUB, _SUB)],
                idx[r][k], lsem[r]).wait()

    def start_scatter(r):
        for k in range(_NSUB):
            pltpu.async_copy(rows[r].at[pl.ds(k * _SUB, _SUB)],
                             acc_s.at[idx[r][k]], ssem[r], add=True)

    def wait_scatter(r):
        for k in range(_NSUB):
            pltpu.make_async_copy(rows[r].at[pl.ds(k * _SUB, _SUB)],
                                  acc_s.at[idx[r][k]], ssem[r]).wait()

    def do_counts(r):
        for k in range(_NSUB):
            for m in range(_SUB // 16):
                iv = idx[r][k][pl.ds(m * 16, 16)]
                plsc.addupdate_scatter(cnt_v, [iv], ones16)

    def step(j, r, has_prev):
        @pl.when(j < jend)
        def _():
            wait_load(r, j)
            rp = (r - 1) % 3
            if has_prev:
                wait_scatter(rp)
            jn = j + 2

            @pl.when(jn < jend)
            def _():
                start_load(rp, jn)

            start_scatter(r)
            do_counts(r)

    for r in range(2):
        jj = jbase + r

        @pl.when(jj < jend)
        def _():
            start_load(r, jj)

    for r in range(3):
        step(jbase + r, r, has_prev=(r != 0))

    def body(i, carry):
        for r in range(3):
            step(jbase + 3 * i + r, r, has_prev=True)
        return carry

    lax.fori_loop(1, _NROUNDS, body, 0)

    for r in range(3):
        @pl.when(lax.rem(nch - 1, 3) == r)
        def _():
            wait_scatter(r)

    plsc.subcore_barrier()
    pltpu.sync_copy(acc_s.at[pl.ds(base, _ROWS_PER_TILE)],
                    psums.at[c, pl.ds(base, _ROWS_PER_TILE)])
    pltpu.sync_copy(cnt_v, pcnts.at[w])


def _combine_body(ps_ref, pc_ref, o_ref):
    sums = ps_ref[0] + ps_ref[1]
    cnts = jnp.sum(pc_ref[...], axis=0)
    denom = jnp.maximum(cnts[:, None], 1.0)
    o_ref[...] = (sums / denom)[:_NUM_CLASSES]


def kernel(context_features, context_labels):
    labels = context_labels.astype(jnp.int32)
    zsum = jnp.zeros((_C_PAD, _D), jnp.float32)
    psums, pcnts = _segment_sums(context_features, labels, zsum)
    return pl.pallas_call(
        _combine_body,
        out_shape=jax.ShapeDtypeStruct((_NUM_CLASSES, _D), jnp.float32),
    )(psums, pcnts)


# speedup vs baseline: 7.3324x; 1.0137x over previous
import functools

import jax
import jax.numpy as jnp
from jax import lax
from jax.experimental import pallas as pl
from jax.experimental.pallas import tpu as pltpu
from jax.experimental.pallas import tpu_sc as plsc

_NUM_CLASSES = 1000
_C_PAD = 1024
_N = 320000
_D = 128
_CHUNK = 256
_SUB = 128
_NSUB = _CHUNK // _SUB
_NCHUNKS = _N // _CHUNK
_NC = 2
_NS = 16
_NW = _NC * _NS
_CPW = _NCHUNKS // _NW
_MAX_CPW = _CPW + 1
_NROUNDS = -(-_MAX_CPW // 3)
_ROWS_PER_TILE = _C_PAD // _NS

_mesh = plsc.VectorSubcoreMesh(core_axis_name="c", subcore_axis_name="s")


@functools.partial(
    pl.kernel,
    out_type=(
        jax.ShapeDtypeStruct((_NC, _C_PAD, _D), jnp.float32),
        jax.ShapeDtypeStruct((_NW, _C_PAD), jnp.float32),
    ),
    mesh=_mesh,
    compiler_params=pltpu.CompilerParams(needs_layout_passes=False),
    scratch_types=[
        pltpu.VMEM((_CHUNK, _D), jnp.float32),
        pltpu.VMEM((_CHUNK, _D), jnp.float32),
        pltpu.VMEM((_CHUNK, _D), jnp.float32),
        pltpu.VMEM((_SUB,), jnp.int32),
        pltpu.VMEM((_SUB,), jnp.int32),
        pltpu.VMEM((_SUB,), jnp.int32),
        pltpu.VMEM((_SUB,), jnp.int32),
        pltpu.VMEM((_SUB,), jnp.int32),
        pltpu.VMEM((_SUB,), jnp.int32),
        pltpu.VMEM((_C_PAD,), jnp.float32),
        pltpu.VMEM_SHARED((_C_PAD, _D), jnp.float32),
        pltpu.SemaphoreType.DMA,
        pltpu.SemaphoreType.DMA,
        pltpu.SemaphoreType.DMA,
        pltpu.SemaphoreType.DMA,
        pltpu.SemaphoreType.DMA,
        pltpu.SemaphoreType.DMA,
    ],
)
def _segment_sums(feat, lab1d, zsum, psums, pcnts,
                  rows0, rows1, rows2,
                  idx00, idx01, idx10, idx11, idx20, idx21,
                  cnt_v, acc_s,
                  lsem0, lsem1, lsem2, ssem0, ssem1, ssem2):
    rows = (rows0, rows1, rows2)
    idx = ((idx00, idx01), (idx10, idx11), (idx20, idx21))
    lsem = (lsem0, lsem1, lsem2)
    ssem = (ssem0, ssem1, ssem2)

    c = lax.axis_index("c")
    s = lax.axis_index("s")
    w = s * _NC + c
    jbase = _CPW * w + jnp.minimum(w, 2)
    jend = _CPW * (w + 1) + jnp.minimum(w + 1, 2)
    nch = jend - jbase

    base = s * _ROWS_PER_TILE
    pltpu.sync_copy(zsum.at[pl.ds(base, _ROWS_PER_TILE)],
                    acc_s.at[pl.ds(base, _ROWS_PER_TILE)])

    zv = jnp.zeros((16,), jnp.float32)

    def zbody(k, carry):
        cnt_v[pl.ds(k * 16, 16)] = zv
        return carry

    lax.fori_loop(0, _C_PAD // 16, zbody, 0)
    plsc.subcore_barrier()

    ones16 = jnp.ones((16,), jnp.float32)

    def start_load(r, j):
        pltpu.async_copy(feat.at[pl.ds(j * _CHUNK, _CHUNK)], rows[r], lsem[r])
        for k in range(_NSUB):
            pltpu.async_copy(lab1d.at[pl.ds(j * _CHUNK + k * _SUB, _SUB)],
                             idx[r][k], lsem[r])

    def wait_load(r, j):
        pltpu.make_async_copy(feat.at[pl.ds(j * _CHUNK, _CHUNK)],
                              rows[r], lsem[r]).wait()
        for k in range(_NSUB):
            pltpu.make_async_copy(
                lab1d.at[pl.ds(j * _CHUNK + k * _SUB, _SUB)],
                idx[r][k], lsem[r]).wait()

    def start_scatter(r):
        for k in range(_NSUB):
            pltpu.async_copy(rows[r].at[pl.ds(k * _SUB, _SUB)],
                             acc_s.at[idx[r][k]], ssem[r], priority=1,
                             add=True)

    def wait_scatter(r):
        for k in range(_NSUB):
            pltpu.make_async_copy(rows[r].at[pl.ds(k * _SUB, _SUB)],
                                  acc_s.at[idx[r][k]], ssem[r]).wait()

    def do_counts(r):
        for k in range(_NSUB):
            for m in range(_SUB // 16):
                iv = idx[r][k][pl.ds(m * 16, 16)]
                plsc.addupdate_scatter(cnt_v, [iv], ones16)

    def step(j, r, has_prev):
        @pl.when(j < jend)
        def _():
            wait_load(r, j)
            start_scatter(r)
            do_counts(r)
            rp = (r - 1) % 3
            if has_prev:
                wait_scatter(rp)
            jn = j + 2

            @pl.when(jn < jend)
            def _():
                start_load(rp, jn)

    for r in range(2):
        jj = jbase + r

        @pl.when(jj < jend)
        def _():
            start_load(r, jj)

    for r in range(3):
        step(jbase + r, r, has_prev=(r != 0))

    def body(i, carry):
        for r in range(3):
            step(jbase + 3 * i + r, r, has_prev=True)
        return carry

    lax.fori_loop(1, _NROUNDS, body, 0)

    for r in range(3):
        @pl.when(lax.rem(nch - 1, 3) == r)
        def _():
            wait_scatter(r)

    plsc.subcore_barrier()
    pltpu.sync_copy(acc_s.at[pl.ds(base, _ROWS_PER_TILE)],
                    psums.at[c, pl.ds(base, _ROWS_PER_TILE)])
    pltpu.sync_copy(cnt_v, pcnts.at[w])


def _combine_body(ps_ref, pc_ref, o_ref):
    sums = ps_ref[0] + ps_ref[1]
    cnts = jnp.sum(pc_ref[...], axis=0)
    denom = jnp.maximum(cnts[:, None], 1.0)
    o_ref[...] = (sums / denom)[:_NUM_CLASSES]


def kernel(context_features, context_labels):
    labels = context_labels.astype(jnp.int32)
    zsum = jnp.zeros((_C_PAD, _D), jnp.float32)
    psums, pcnts = _segment_sums(context_features, labels, zsum)
    return pl.pallas_call(
        _combine_body,
        out_shape=jax.ShapeDtypeStruct((_NUM_CLASSES, _D), jnp.float32),
    )(psums, pcnts)


# speedup vs baseline: 7.3351x; 1.0004x over previous
import functools

import jax
import jax.numpy as jnp
from jax import lax
from jax.experimental import pallas as pl
from jax.experimental.pallas import tpu as pltpu
from jax.experimental.pallas import tpu_sc as plsc

_NUM_CLASSES = 1000
_C_PAD = 1024
_N = 320000
_D = 128
_CHUNK = 256
_SUB = 128
_NSUB = _CHUNK // _SUB
_NCHUNKS = _N // _CHUNK
_NC = 2
_NS = 16
_NW = _NC * _NS
_CPW = _NCHUNKS // _NW
_MAX_CPW = _CPW + 1
_NROUNDS = -(-_MAX_CPW // 3)
_ROWS_PER_TILE = _C_PAD // _NS

_mesh = plsc.VectorSubcoreMesh(core_axis_name="c", subcore_axis_name="s")


@functools.partial(
    pl.kernel,
    out_type=(
        jax.ShapeDtypeStruct((_NC, _C_PAD, _D), jnp.float32),
        jax.ShapeDtypeStruct((_NW, _C_PAD), jnp.float32),
    ),
    mesh=_mesh,
    compiler_params=pltpu.CompilerParams(needs_layout_passes=False),
    scratch_types=[
        pltpu.VMEM((_CHUNK, _D), jnp.float32),
        pltpu.VMEM((_CHUNK, _D), jnp.float32),
        pltpu.VMEM((_CHUNK, _D), jnp.float32),
        pltpu.VMEM((_SUB,), jnp.int32),
        pltpu.VMEM((_SUB,), jnp.int32),
        pltpu.VMEM((_SUB,), jnp.int32),
        pltpu.VMEM((_SUB,), jnp.int32),
        pltpu.VMEM((_SUB,), jnp.int32),
        pltpu.VMEM((_SUB,), jnp.int32),
        pltpu.VMEM((_C_PAD,), jnp.float32),
        pltpu.VMEM_SHARED((_C_PAD, _D), jnp.float32),
        pltpu.SemaphoreType.DMA,
        pltpu.SemaphoreType.DMA,
        pltpu.SemaphoreType.DMA,
        pltpu.SemaphoreType.DMA,
        pltpu.SemaphoreType.DMA,
        pltpu.SemaphoreType.DMA,
    ],
)
def _segment_sums(feat, lab1d, zsum, psums, pcnts,
                  rows0, rows1, rows2,
                  idx00, idx01, idx10, idx11, idx20, idx21,
                  cnt_v, acc_s,
                  lsem0, lsem1, lsem2, ssem0, ssem1, ssem2):
    rows = (rows0, rows1, rows2)
    idx = ((idx00, idx01), (idx10, idx11), (idx20, idx21))
    lsem = (lsem0, lsem1, lsem2)
    ssem = (ssem0, ssem1, ssem2)

    c = lax.axis_index("c")
    s = lax.axis_index("s")
    w = s * _NC + c
    jbase = _CPW * w + jnp.minimum(w, 2)
    jend = _CPW * (w + 1) + jnp.minimum(w + 1, 2)
    nch = jend - jbase

    base = s * _ROWS_PER_TILE
    pltpu.sync_copy(zsum.at[pl.ds(base, _ROWS_PER_TILE)],
                    acc_s.at[pl.ds(base, _ROWS_PER_TILE)])

    zv = jnp.zeros((16,), jnp.float32)

    def zbody(k, carry):
        cnt_v[pl.ds(k * 16, 16)] = zv
        return carry

    lax.fori_loop(0, _C_PAD // 16, zbody, 0)
    plsc.subcore_barrier()

    ones16 = jnp.ones((16,), jnp.float32)

    def start_load(r, j):
        pltpu.async_copy(feat.at[pl.ds(j * _CHUNK, _CHUNK)], rows[r], lsem[r])
        for k in range(_NSUB):
            pltpu.async_copy(lab1d.at[pl.ds(j * _CHUNK + k * _SUB, _SUB)],
                             idx[r][k], lsem[r])

    def wait_load(r, j):
        pltpu.make_async_copy(feat.at[pl.ds(j * _CHUNK, _CHUNK)],
                              rows[r], lsem[r]).wait()
        for k in range(_NSUB):
            pltpu.make_async_copy(
                lab1d.at[pl.ds(j * _CHUNK + k * _SUB, _SUB)],
                idx[r][k], lsem[r]).wait()

    def start_scatter(r):
        for k in range(_NSUB):
            pltpu.async_copy(rows[r].at[pl.ds(k * _SUB, _SUB)],
                             acc_s.at[idx[r][k]], ssem[r], add=True)

    def wait_scatter(r):
        for k in range(_NSUB):
            pltpu.make_async_copy(rows[r].at[pl.ds(k * _SUB, _SUB)],
                                  acc_s.at[idx[r][k]], ssem[r]).wait()

    def do_counts(r):
        for k in range(_NSUB):
            for m in range(_SUB // 16):
                iv = idx[r][k][pl.ds(m * 16, 16)]
                plsc.addupdate_scatter(cnt_v, [iv], ones16)

    def step(j, r, has_prev):
        @pl.when(j < jend)
        def _():
            wait_load(r, j)
            start_scatter(r)
            do_counts(r)
            rp = (r - 1) % 3
            if has_prev:
                wait_scatter(rp)
            jn = j + 2

            @pl.when(jn < jend)
            def _():
                start_load(rp, jn)

    for r in range(2):
        jj = jbase + r

        @pl.when(jj < jend)
        def _():
            start_load(r, jj)

    for r in range(3):
        step(jbase + r, r, has_prev=(r != 0))

    def body(i, carry):
        for r in range(3):
            step(jbase + 3 * i + r, r, has_prev=True)
        return carry

    lax.fori_loop(1, _NROUNDS, body, 0)

    for r in range(3):
        @pl.when(lax.rem(nch - 1, 3) == r)
        def _():
            wait_scatter(r)

    plsc.subcore_barrier()
    pltpu.sync_copy(acc_s.at[pl.ds(base, _ROWS_PER_TILE)],
                    psums.at[c, pl.ds(base, _ROWS_PER_TILE)])
    pltpu.sync_copy(cnt_v, pcnts.at[w])


def _combine_body(ps_ref, pc_ref, o_ref):
    sums = ps_ref[0] + ps_ref[1]
    cnts = jnp.sum(pc_ref[...], axis=0)
    denom = jnp.maximum(cnts[:, None], 1.0)
    o_ref[...] = (sums / denom)[:_NUM_CLASSES]


def kernel(context_features, context_labels):
    labels = context_labels.astype(jnp.int32)
    zsum = jnp.zeros((_C_PAD, _D), jnp.float32)
    psums, pcnts = _segment_sums(context_features, labels, zsum)
    return pl.pallas_call(
        _combine_body,
        out_shape=jax.ShapeDtypeStruct((_NUM_CLASSES, _D), jnp.float32),
    )(psums, pcnts)
